# Initial kernel scaffold; baseline (speedup 1.0000x reference)
import jax
import jax.numpy as jnp
from jax.experimental import pallas as pl


def kernel(x, h, edge_index, edge_fea, em_W1, em_b1, em_W2, em_b2, co_W1, co_b1, co_W2, co_b2, nn_W1, nn_b1, nn_W2, nn_b2):
    raise NotImplementedError("write your pallas kernel here")


# speedup vs baseline: 3.0059x; 3.0059x over previous
import functools

import jax
import jax.numpy as jnp
from jax import lax
from jax.experimental import pallas as pl
from jax.experimental.pallas import tpu as pltpu
from jax.experimental.pallas import tpu_sc as plsc

_NC = 2
_NS = 16
_NW = _NC * _NS
_CH = 80
_PW = 16
_L = 16


def _silu(v):
    return v * jax.nn.sigmoid(v)


def _tables_body(x_ref, h_ref, w1r_ref, w1c_ref, b1_ref, a_ref, b_ref):
    hb = h_ref[...]
    xb = x_ref[...]
    xpad = jnp.concatenate(
        [xb, jnp.zeros((xb.shape[0], 128 - xb.shape[1]), jnp.float32)], axis=1)
    a_ref[:, 0:128] = jnp.dot(hb, w1r_ref[...], preferred_element_type=jnp.float32) + b1_ref[...]
    a_ref[:, 128:256] = xpad
    b_ref[:, 0:128] = jnp.dot(hb, w1c_ref[...], preferred_element_type=jnp.float32)
    b_ref[:, 128:256] = -xpad


def _build_tables(x, h, w1r, w1c, b1):
    n, hdim = h.shape
    bn = 1000
    assert n % bn == 0
    return pl.pallas_call(
        _tables_body,
        grid=(n // bn,),
        in_specs=[
            pl.BlockSpec((bn, x.shape[1]), lambda i: (i, 0)),
            pl.BlockSpec((bn, hdim), lambda i: (i, 0)),
            pl.BlockSpec(w1r.shape, lambda i: (0, 0)),
            pl.BlockSpec(w1c.shape, lambda i: (0, 0)),
            pl.BlockSpec(b1.shape, lambda i: (0, 0)),
        ],
        out_specs=[pl.BlockSpec((bn, 256), lambda i: (i, 0))] * 2,
        out_shape=[jax.ShapeDtypeStruct((n, 256), jnp.float32)] * 2,
    )(x, h, w1r, w1c, b1)


def _sc_gather(a_tab, b_tab, row, col):
    e = row.shape[0]
    w = 128 + _PW
    assert e % (_NW * _CH) == 0
    ew = e // _NW
    n_ch = ew // _CH
    mesh = plsc.VectorSubcoreMesh(
        core_axis_name="c", subcore_axis_name="s",
        num_cores=_NC, num_subcores=_NS)

    @functools.partial(
        pl.kernel,
        out_type=jax.ShapeDtypeStruct((e, w), jnp.float32),
        mesh=mesh,
        scratch_types=[
            pltpu.VMEM((_CH,), jnp.int32),
            pltpu.VMEM((_CH,), jnp.int32),
            pltpu.VMEM((_CH, 256), jnp.float32),
            pltpu.VMEM((_CH, 256), jnp.float32),
            pltpu.VMEM((_CH, w), jnp.float32),
            pltpu.SemaphoreType.DMA,
            pltpu.SemaphoreType.DMA,
        ],
    )
    def gather_kernel(a_hbm, b_hbm, row_hbm, col_hbm, p_hbm,
                      ridx, cidx, buf1, buf2, pbuf, sem1, sem2):
        wid = lax.axis_index("s") * _NC + lax.axis_index("c")
        base = wid * ew

        def body(i, carry):
            e0 = base + i * _CH
            pltpu.sync_copy(row_hbm.at[pl.ds(e0, _CH)], ridx)
            pltpu.sync_copy(col_hbm.at[pl.ds(e0, _CH)], cidx)
            cp1 = pltpu.async_copy(a_hbm.at[ridx], buf1, sem1)
            cp2 = pltpu.async_copy(b_hbm.at[cidx], buf2, sem2)
            cp1.wait()
            cp2.wait()

            def sum_row(r, c2):
                for cc in range(w // _L):
                    sl = pl.ds(cc * _L, _L)
                    pbuf[r, sl] = buf1[r, sl] + buf2[r, sl]
                return c2

            lax.fori_loop(0, _CH, sum_row, 0)
            pltpu.sync_copy(pbuf, p_hbm.at[pl.ds(e0, _CH)])
            return carry

        lax.fori_loop(0, n_ch, body, 0)

    return gather_kernel(a_tab, b_tab, row, col)


def _edge_body(p_ref, ef_ref, w1e_ref, w1s_ref, w2_ref, b2_ref,
               cw1_ref, cb1_ref, cw2_ref, cb2_ref, msg_ref, faux_ref):
    p = p_ref[...]
    rij = p[:, 128:144]
    scal = jnp.sum(rij * rij, axis=1, keepdims=True)
    pre1 = (p[:, 0:128]
            + jnp.dot(ef_ref[...], w1e_ref[...], preferred_element_type=jnp.float32)
            + scal * w1s_ref[...])
    u = _silu(pre1)
    msg = _silu(jnp.dot(u, w2_ref[...], preferred_element_type=jnp.float32)
                + b2_ref[...])
    t = _silu(jnp.dot(msg, cw1_ref[...], preferred_element_type=jnp.float32)
              + cb1_ref[...])
    cm = jnp.dot(t, cw2_ref[...], preferred_element_type=jnp.float32) + cb2_ref[...]
    lane = lax.broadcasted_iota(jnp.int32, rij.shape, 1)
    faux = jnp.where(lane == 3, 1.0, rij * cm)
    msg_ref[...] = msg
    faux_ref[...] = faux


def _edge_mlp(p, edge_fea, w1e, w1s, w2, b2, cw1, cb1, cw2, cb2):
    e, w = p.shape
    be = 512
    assert e % be == 0
    full = lambda a: pl.BlockSpec(a.shape, lambda i: (0,) * a.ndim)
    return pl.pallas_call(
        _edge_body,
        grid=(e // be,),
        in_specs=[
            pl.BlockSpec((be, w), lambda i: (i, 0)),
            pl.BlockSpec((be, edge_fea.shape[1]), lambda i: (i, 0)),
            full(w1e), full(w1s), full(w2), full(b2),
            full(cw1), full(cb1), full(cw2), full(cb2),
        ],
        out_specs=[
            pl.BlockSpec((be, 128), lambda i: (i, 0)),
            pl.BlockSpec((be, _PW), lambda i: (i, 0)),
        ],
        out_shape=[
            jax.ShapeDtypeStruct((e, 128), jnp.float32),
            jax.ShapeDtypeStruct((e, _PW), jnp.float32),
        ],
    )(p, edge_fea, w1e, w1s, w2, b2, cw1, cb1, cw2, cb2)


def _sc_scatter(row, msg, faux, n):
    e = row.shape[0]
    npad = ((n + _NS * _CH - 1) // (_NS * _CH)) * (_NS * _CH)
    ept = e // _NS
    assert e % (_NS * _CH) == 0
    n_ch = ept // _CH
    rpt = npad // _NS
    assert rpt % _CH == 0
    nzc = rpt // _CH
    mesh = plsc.VectorSubcoreMesh(
        core_axis_name="c", subcore_axis_name="s",
        num_cores=_NC, num_subcores=_NS)

    @functools.partial(
        pl.kernel,
        out_type=[
            jax.ShapeDtypeStruct((npad, 128), jnp.float32),
            jax.ShapeDtypeStruct((npad, 128), jnp.float32),
        ],
        mesh=mesh,
        scratch_types=[
            pltpu.VMEM((_CH,), jnp.int32),
            pltpu.VMEM((_CH, 128), jnp.float32),
            pltpu.VMEM((_CH, _PW), jnp.float32),
            pltpu.VMEM((_CH, 128), jnp.float32),
            pltpu.VMEM_SHARED((npad, 128), jnp.float32),
        ],
    )
    def scatter_kernel(row_hbm, msg_hbm, faux_hbm, mp_hbm, fp_hbm,
                       idx, mbuf, fbuf, f128, acc):
        c = lax.axis_index("c")
        s = lax.axis_index("s")
        r0 = s * rpt
        base = s * ept
        zero16 = jnp.zeros((_L,), jnp.float32)

        def zrow(r, c2):
            for cc in range(128 // _L):
                f128[r, pl.ds(cc * _L, _L)] = zero16
            return c2

        lax.fori_loop(0, _CH, zrow, 0)
        for k in range(nzc):
            pltpu.sync_copy(f128, acc.at[pl.ds(r0 + k * _CH, _CH)])
        plsc.subcore_barrier()

        @pl.when(c == 0)
        def _msg_loop():
            def body(i, carry):
                e0 = base + i * _CH
                pltpu.sync_copy(row_hbm.at[pl.ds(e0, _CH)], idx)
                pltpu.sync_copy(msg_hbm.at[pl.ds(e0, _CH)], mbuf)
                pltpu.sync_copy(mbuf, acc.at[idx], add=True)
                return carry

            lax.fori_loop(0, n_ch, body, 0)

        @pl.when(c == 1)
        def _faux_loop():
            def body(i, carry):
                e0 = base + i * _CH
                pltpu.sync_copy(row_hbm.at[pl.ds(e0, _CH)], idx)
                pltpu.sync_copy(faux_hbm.at[pl.ds(e0, _CH)], fbuf)

                def frow(r, c2):
                    f128[r, pl.ds(0, _PW)] = fbuf[r, :]
                    return c2

                lax.fori_loop(0, _CH, frow, 0)
                pltpu.sync_copy(f128, acc.at[idx], add=True)
                return carry

            lax.fori_loop(0, n_ch, body, 0)

        plsc.subcore_barrier()

        for k in range(nzc):
            pltpu.sync_copy(acc.at[pl.ds(r0 + k * _CH, _CH)], mbuf)

            @pl.when(c == 0)
            def _wm():
                pltpu.sync_copy(mbuf, mp_hbm.at[pl.ds(r0 + k * _CH, _CH)])

            @pl.when(c == 1)
            def _wf():
                pltpu.sync_copy(mbuf, fp_hbm.at[pl.ds(r0 + k * _CH, _CH)])

    return scatter_kernel(row, msg, faux)


def _node_body(x_ref, h_ref, mp0_ref, fp0_ref,
               w1a_ref, w1b_ref, b1_ref, w2_ref, b2_ref, xo_ref, ho_ref):
    tm = mp0_ref[...]
    tf16 = fp0_ref[...]
    deg = tf16[:, 3:4]
    tf = jnp.clip(tf16[:, 0:3] / jnp.maximum(deg, 1.0), -100.0, 100.0)
    xo_ref[...] = x_ref[...] + tf
    z = _silu(jnp.dot(h_ref[...], w1a_ref[...], preferred_element_type=jnp.float32)
              + jnp.dot(tm, w1b_ref[...], preferred_element_type=jnp.float32)
              + b1_ref[...])
    ho_ref[...] = jnp.dot(z, w2_ref[...], preferred_element_type=jnp.float32) + b2_ref[...]


def _node_mlp(x, h, mp0, fp0, w1a, w1b, b1, w2, b2):
    n, hdim = h.shape
    bn = 1000
    assert n % bn == 0
    full = lambda a: pl.BlockSpec(a.shape, lambda i: (0,) * a.ndim)
    return pl.pallas_call(
        _node_body,
        grid=(n // bn,),
        in_specs=[
            pl.BlockSpec((bn, x.shape[1]), lambda i: (i, 0)),
            pl.BlockSpec((bn, hdim), lambda i: (i, 0)),
            pl.BlockSpec((bn, 128), lambda i: (i, 0)),
            pl.BlockSpec((bn, 128), lambda i: (i, 0)),
            full(w1a), full(w1b), full(b1), full(w2), full(b2),
        ],
        out_specs=[
            pl.BlockSpec((bn, x.shape[1]), lambda i: (i, 0)),
            pl.BlockSpec((bn, hdim), lambda i: (i, 0)),
        ],
        out_shape=[
            jax.ShapeDtypeStruct((n, x.shape[1]), jnp.float32),
            jax.ShapeDtypeStruct((n, hdim), jnp.float32),
        ],
    )(x, h, mp0, fp0, w1a, w1b, b1, w2, b2)


def kernel(x, h, edge_index, edge_fea,
           em_W1, em_b1, em_W2, em_b2,
           co_W1, co_b1, co_W2, co_b2,
           nn_W1, nn_b1, nn_W2, nn_b2):
    n, hdim = h.shape
    row = edge_index[0].astype(jnp.int32)
    col = edge_index[1].astype(jnp.int32)

    w1s = em_W1[0:1, :]
    w1r = em_W1[1:1 + hdim, :]
    w1c = em_W1[1 + hdim:1 + 2 * hdim, :]
    w1e = em_W1[1 + 2 * hdim:, :]

    a_tab, b_tab = _build_tables(x, h, w1r, w1c, em_b1.reshape(1, -1))
    p = _sc_gather(a_tab, b_tab, row, col)
    msg, faux = _edge_mlp(
        p, edge_fea, w1e, w1s,
        em_W2, em_b2.reshape(1, -1),
        co_W1, co_b1.reshape(1, -1), co_W2, co_b2.reshape(1, 1))
    mp, fp = _sc_scatter(row, msg, faux, n)
    x_new, h_new = _node_mlp(
        x, h, mp[:n], fp[:n],
        nn_W1[:hdim], nn_W1[hdim:], nn_b1.reshape(1, -1),
        nn_W2, nn_b2.reshape(1, -1))
    return (x_new, h_new)


# speedup vs baseline: 3.9473x; 1.3132x over previous
import functools

import jax
import jax.numpy as jnp
from jax import lax
from jax.experimental import pallas as pl
from jax.experimental.pallas import tpu as pltpu
from jax.experimental.pallas import tpu_sc as plsc

_NC = 2
_NS = 16
_NW = _NC * _NS
_CH = 80
_PW = 16
_L = 16
_CHS = 40


def _silu(v):
    return v * jax.nn.sigmoid(v)


def _tables_body(x_ref, h_ref, w1r_ref, w1c_ref, b1_ref, a_ref, b_ref):
    hb = h_ref[...]
    xb = x_ref[...]
    xpad = jnp.concatenate(
        [xb, jnp.zeros((xb.shape[0], 128 - xb.shape[1]), jnp.float32)], axis=1)
    a_ref[:, 0:128] = jnp.dot(hb, w1r_ref[...], preferred_element_type=jnp.float32) + b1_ref[...]
    a_ref[:, 128:256] = xpad
    b_ref[:, 0:128] = jnp.dot(hb, w1c_ref[...], preferred_element_type=jnp.float32)
    b_ref[:, 128:256] = -xpad


def _build_tables(x, h, w1r, w1c, b1):
    n, hdim = h.shape
    bn = 1000
    assert n % bn == 0
    return pl.pallas_call(
        _tables_body,
        grid=(n // bn,),
        in_specs=[
            pl.BlockSpec((bn, x.shape[1]), lambda i: (i, 0)),
            pl.BlockSpec((bn, hdim), lambda i: (i, 0)),
            pl.BlockSpec(w1r.shape, lambda i: (0, 0)),
            pl.BlockSpec(w1c.shape, lambda i: (0, 0)),
            pl.BlockSpec(b1.shape, lambda i: (0, 0)),
        ],
        out_specs=[pl.BlockSpec((bn, 256), lambda i: (i, 0))] * 2,
        out_shape=[jax.ShapeDtypeStruct((n, 256), jnp.float32)] * 2,
    )(x, h, w1r, w1c, b1)


def _sc_gather(a_tab, b_tab, row, col):
    e = row.shape[0]
    w = 128 + _PW
    assert e % (_NW * _CH) == 0
    ew = e // _NW
    n_ch = ew // _CH
    assert n_ch % 2 == 1 and n_ch >= 3
    mesh = plsc.VectorSubcoreMesh(
        core_axis_name="c", subcore_axis_name="s",
        num_cores=_NC, num_subcores=_NS)

    @functools.partial(
        pl.kernel,
        out_type=jax.ShapeDtypeStruct((e, w), jnp.float32),
        mesh=mesh,
        scratch_types=[
            pltpu.VMEM((_CH,), jnp.int32), pltpu.VMEM((_CH,), jnp.int32),
            pltpu.VMEM((_CH,), jnp.int32), pltpu.VMEM((_CH,), jnp.int32),
            pltpu.VMEM((_CH, 256), jnp.float32), pltpu.VMEM((_CH, 256), jnp.float32),
            pltpu.VMEM((_CH, 256), jnp.float32), pltpu.VMEM((_CH, 256), jnp.float32),
            pltpu.VMEM((_CH, w), jnp.float32), pltpu.VMEM((_CH, w), jnp.float32),
            pltpu.SemaphoreType.DMA, pltpu.SemaphoreType.DMA,
            pltpu.SemaphoreType.DMA, pltpu.SemaphoreType.DMA,
            pltpu.SemaphoreType.DMA, pltpu.SemaphoreType.DMA,
        ],
    )
    def gather_kernel(a_hbm, b_hbm, row_hbm, col_hbm, p_hbm,
                      ridx0, cidx0, ridx1, cidx1,
                      a0, b0, a1, b1, pb0, pb1,
                      si0, si1, sg0, sg1, so0, so1):
        wid = lax.axis_index("s") * _NC + lax.axis_index("c")
        base = wid * ew
        ridx = (ridx0, ridx1)
        cidx = (cidx0, cidx1)
        ab = ((a0, b0), (a1, b1))
        pb = (pb0, pb1)
        si = (si0, si1)
        sg = (sg0, sg1)
        so = (so0, so1)

        def start_idx(slot, ci):
            e0 = base + ci * _CH
            pltpu.async_copy(row_hbm.at[pl.ds(e0, _CH)], ridx[slot], si[slot])
            pltpu.async_copy(col_hbm.at[pl.ds(e0, _CH)], cidx[slot], si[slot])

        def wait_idx(slot):
            pltpu.make_async_copy(row_hbm.at[pl.ds(0, _CH)], ridx[slot], si[slot]).wait()
            pltpu.make_async_copy(col_hbm.at[pl.ds(0, _CH)], cidx[slot], si[slot]).wait()

        def start_gather(slot):
            pltpu.async_copy(a_hbm.at[ridx[slot]], ab[slot][0], sg[slot])
            pltpu.async_copy(b_hbm.at[cidx[slot]], ab[slot][1], sg[slot])

        def wait_gather(slot):
            pltpu.make_async_copy(a_hbm.at[ridx[slot]], ab[slot][0], sg[slot]).wait()
            pltpu.make_async_copy(b_hbm.at[cidx[slot]], ab[slot][1], sg[slot]).wait()

        def vadd(slot):
            b1_, b2_ = ab[slot]
            pbuf = pb[slot]

            def sum_row(r, c2):
                for cc in range(w // _L):
                    sl = pl.ds(cc * _L, _L)
                    pbuf[r, sl] = b1_[r, sl] + b2_[r, sl]
                return c2

            lax.fori_loop(0, _CH, sum_row, 0)

        def start_out(slot, ci):
            e0 = base + ci * _CH
            pltpu.async_copy(pb[slot], p_hbm.at[pl.ds(e0, _CH)], so[slot])

        def wait_out(slot):
            pltpu.make_async_copy(pb[slot], p_hbm.at[pl.ds(0, _CH)], so[slot]).wait()

        start_idx(0, 0)
        start_idx(1, 1)
        wait_idx(0)
        start_gather(0)

        def body(k, carry):
            c0 = 2 * k
            wait_idx(1)
            start_gather(1)
            wait_gather(0)
            start_idx(0, c0 + 2)

            @pl.when(k > 0)
            def _():
                wait_out(0)

            vadd(0)
            start_out(0, c0)
            wait_idx(0)
            start_gather(0)
            wait_gather(1)

            @pl.when(k < (n_ch - 3) // 2)
            def _():
                start_idx(1, c0 + 3)

            @pl.when(k > 0)
            def _():
                wait_out(1)

            vadd(1)
            start_out(1, c0 + 1)
            return carry

        lax.fori_loop(0, (n_ch - 1) // 2, body, 0)
        wait_gather(0)
        wait_out(0)
        vadd(0)
        start_out(0, n_ch - 1)
        wait_out(0)
        wait_out(1)

    return gather_kernel(a_tab, b_tab, row, col)


def _edge_body(p_ref, ef_ref, w1e_ref, w1s_ref, w2_ref, b2_ref,
               cw1_ref, cb1_ref, cw2_ref, cb2_ref, msg_ref, faux_ref):
    p = p_ref[...]
    rij = p[:, 128:144]
    scal = jnp.sum(rij * rij, axis=1, keepdims=True)
    pre1 = (p[:, 0:128]
            + jnp.dot(ef_ref[...], w1e_ref[...], preferred_element_type=jnp.float32)
            + scal * w1s_ref[...])
    u = _silu(pre1)
    msg = _silu(jnp.dot(u, w2_ref[...], preferred_element_type=jnp.float32)
                + b2_ref[...])
    t = _silu(jnp.dot(msg, cw1_ref[...], preferred_element_type=jnp.float32)
              + cb1_ref[...])
    cm = jnp.dot(t, cw2_ref[...], preferred_element_type=jnp.float32) + cb2_ref[...]
    lane = lax.broadcasted_iota(jnp.int32, rij.shape, 1)
    faux = jnp.where(lane == 3, 1.0, rij * cm)
    msg_ref[...] = msg
    faux_ref[...] = faux


def _edge_mlp(p, edge_fea, w1e, w1s, w2, b2, cw1, cb1, cw2, cb2):
    e, w = p.shape
    be = 512
    assert e % be == 0
    full = lambda a: pl.BlockSpec(a.shape, lambda i: (0,) * a.ndim)
    return pl.pallas_call(
        _edge_body,
        grid=(e // be,),
        in_specs=[
            pl.BlockSpec((be, w), lambda i: (i, 0)),
            pl.BlockSpec((be, edge_fea.shape[1]), lambda i: (i, 0)),
            full(w1e), full(w1s), full(w2), full(b2),
            full(cw1), full(cb1), full(cw2), full(cb2),
        ],
        out_specs=[
            pl.BlockSpec((be, 128), lambda i: (i, 0)),
            pl.BlockSpec((be, _PW), lambda i: (i, 0)),
        ],
        out_shape=[
            jax.ShapeDtypeStruct((e, 128), jnp.float32),
            jax.ShapeDtypeStruct((e, _PW), jnp.float32),
        ],
    )(p, edge_fea, w1e, w1s, w2, b2, cw1, cb1, cw2, cb2)


def _sc_scatter(row, msg, faux, n):
    e = row.shape[0]
    npad = ((n + _NS * _CHS - 1) // (_NS * _CHS)) * (_NS * _CHS)
    ept = e // _NS
    assert e % (_NS * _CHS) == 0
    n_ch = ept // _CHS
    assert n_ch % 2 == 0
    rpt = npad // _NS
    assert rpt % _CHS == 0
    nzc = rpt // _CHS
    mesh = plsc.VectorSubcoreMesh(
        core_axis_name="c", subcore_axis_name="s",
        num_cores=_NC, num_subcores=_NS)

    @functools.partial(
        pl.kernel,
        out_type=[
            jax.ShapeDtypeStruct((npad, 128), jnp.float32),
            jax.ShapeDtypeStruct((npad, 128), jnp.float32),
        ],
        mesh=mesh,
        scratch_types=[
            pltpu.VMEM((_CHS,), jnp.int32), pltpu.VMEM((_CHS,), jnp.int32),
            pltpu.VMEM((_CHS, 128), jnp.float32), pltpu.VMEM((_CHS, 128), jnp.float32),
            pltpu.VMEM((_CHS, _PW), jnp.float32), pltpu.VMEM((_CHS, _PW), jnp.float32),
            pltpu.VMEM((_CHS, 128), jnp.float32), pltpu.VMEM((_CHS, 128), jnp.float32),
            pltpu.VMEM_SHARED((npad, 128), jnp.float32),
            pltpu.SemaphoreType.DMA, pltpu.SemaphoreType.DMA,
            pltpu.SemaphoreType.DMA, pltpu.SemaphoreType.DMA,
        ],
    )
    def scatter_kernel(row_hbm, msg_hbm, faux_hbm, mp_hbm, fp_hbm,
                       idx0, idx1, mbuf0, mbuf1, fbuf0, fbuf1, f0, f1, acc,
                       sl0, sl1, ss0, ss1):
        c = lax.axis_index("c")
        s = lax.axis_index("s")
        r0 = s * rpt
        base = s * ept
        zero16 = jnp.zeros((_L,), jnp.float32)
        idx = (idx0, idx1)
        mbuf = (mbuf0, mbuf1)
        fbuf = (fbuf0, fbuf1)
        f128 = (f0, f1)
        sl = (sl0, sl1)
        ss = (ss0, ss1)

        def zrow(r, c2):
            for cc in range(128 // _L):
                f0[r, pl.ds(cc * _L, _L)] = zero16
                f1[r, pl.ds(cc * _L, _L)] = zero16
            return c2

        lax.fori_loop(0, _CHS, zrow, 0)
        for k in range(nzc):
            pltpu.sync_copy(f0, acc.at[pl.ds(r0 + k * _CHS, _CHS)])
        plsc.subcore_barrier()

        def start_load(slot, ci, payload_hbm, payload_buf):
            e0 = base + ci * _CHS
            pltpu.async_copy(row_hbm.at[pl.ds(e0, _CHS)], idx[slot], sl[slot])
            pltpu.async_copy(payload_hbm.at[pl.ds(e0, _CHS)], payload_buf[slot], sl[slot])

        def wait_load(slot, payload_hbm, payload_buf):
            pltpu.make_async_copy(row_hbm.at[pl.ds(0, _CHS)], idx[slot], sl[slot]).wait()
            pltpu.make_async_copy(payload_hbm.at[pl.ds(0, _CHS)], payload_buf[slot], sl[slot]).wait()

        def start_scat(slot, payload_buf):
            pltpu.async_copy(payload_buf[slot], acc.at[idx[slot]], ss[slot], add=True)

        def wait_scat(slot, payload_buf):
            pltpu.make_async_copy(payload_buf[slot], acc.at[idx[slot]], ss[slot]).wait()

        @pl.when(c == 0)
        def _msg_loop():
            start_load(0, 0, msg_hbm, mbuf)
            start_load(1, 1, msg_hbm, mbuf)

            def body(k, carry):
                c0 = 2 * k
                wait_load(0, msg_hbm, mbuf)
                start_scat(0, mbuf)
                wait_load(1, msg_hbm, mbuf)
                start_scat(1, mbuf)
                wait_scat(0, mbuf)

                @pl.when(k < n_ch // 2 - 1)
                def _():
                    start_load(0, c0 + 2, msg_hbm, mbuf)

                wait_scat(1, mbuf)

                @pl.when(k < n_ch // 2 - 1)
                def _():
                    start_load(1, c0 + 3, msg_hbm, mbuf)

                return carry

            lax.fori_loop(0, n_ch // 2, body, 0)

        @pl.when(c == 1)
        def _faux_loop():
            start_load(0, 0, faux_hbm, fbuf)
            start_load(1, 1, faux_hbm, fbuf)

            def fcopy(slot):
                fb = fbuf[slot]
                fw = f128[slot]

                def frow(r, c2):
                    fw[r, pl.ds(0, _PW)] = fb[r, :]
                    return c2

                lax.fori_loop(0, _CHS, frow, 0)

            def body(k, carry):
                c0 = 2 * k
                wait_load(0, faux_hbm, fbuf)
                fcopy(0)
                start_scat(0, f128)
                wait_load(1, faux_hbm, fbuf)
                fcopy(1)
                start_scat(1, f128)
                wait_scat(0, f128)

                @pl.when(k < n_ch // 2 - 1)
                def _():
                    start_load(0, c0 + 2, faux_hbm, fbuf)

                wait_scat(1, f128)

                @pl.when(k < n_ch // 2 - 1)
                def _():
                    start_load(1, c0 + 3, faux_hbm, fbuf)

                return carry

            lax.fori_loop(0, n_ch // 2, body, 0)

        plsc.subcore_barrier()

        for k in range(nzc):
            pltpu.sync_copy(acc.at[pl.ds(r0 + k * _CHS, _CHS)], mbuf0)

            @pl.when(c == 0)
            def _wm():
                pltpu.sync_copy(mbuf0, mp_hbm.at[pl.ds(r0 + k * _CHS, _CHS)])

            @pl.when(c == 1)
            def _wf():
                pltpu.sync_copy(mbuf0, fp_hbm.at[pl.ds(r0 + k * _CHS, _CHS)])

    return scatter_kernel(row, msg, faux)


def _node_body(x_ref, h_ref, mp0_ref, fp0_ref,
               w1a_ref, w1b_ref, b1_ref, w2_ref, b2_ref, xo_ref, ho_ref):
    tm = mp0_ref[...]
    tf16 = fp0_ref[...]
    deg = tf16[:, 3:4]
    tf = jnp.clip(tf16[:, 0:3] / jnp.maximum(deg, 1.0), -100.0, 100.0)
    xo_ref[...] = x_ref[...] + tf
    z = _silu(jnp.dot(h_ref[...], w1a_ref[...], preferred_element_type=jnp.float32)
              + jnp.dot(tm, w1b_ref[...], preferred_element_type=jnp.float32)
              + b1_ref[...])
    ho_ref[...] = jnp.dot(z, w2_ref[...], preferred_element_type=jnp.float32) + b2_ref[...]


def _node_mlp(x, h, mp0, fp0, w1a, w1b, b1, w2, b2):
    n, hdim = h.shape
    bn = 1000
    assert n % bn == 0
    full = lambda a: pl.BlockSpec(a.shape, lambda i: (0,) * a.ndim)
    return pl.pallas_call(
        _node_body,
        grid=(n // bn,),
        in_specs=[
            pl.BlockSpec((bn, x.shape[1]), lambda i: (i, 0)),
            pl.BlockSpec((bn, hdim), lambda i: (i, 0)),
            pl.BlockSpec((bn, 128), lambda i: (i, 0)),
            pl.BlockSpec((bn, 128), lambda i: (i, 0)),
            full(w1a), full(w1b), full(b1), full(w2), full(b2),
        ],
        out_specs=[
            pl.BlockSpec((bn, x.shape[1]), lambda i: (i, 0)),
            pl.BlockSpec((bn, hdim), lambda i: (i, 0)),
        ],
        out_shape=[
            jax.ShapeDtypeStruct((n, x.shape[1]), jnp.float32),
            jax.ShapeDtypeStruct((n, hdim), jnp.float32),
        ],
    )(x, h, mp0, fp0, w1a, w1b, b1, w2, b2)


def kernel(x, h, edge_index, edge_fea,
           em_W1, em_b1, em_W2, em_b2,
           co_W1, co_b1, co_W2, co_b2,
           nn_W1, nn_b1, nn_W2, nn_b2):
    n, hdim = h.shape
    row = edge_index[0].astype(jnp.int32)
    col = edge_index[1].astype(jnp.int32)

    w1s = em_W1[0:1, :]
    w1r = em_W1[1:1 + hdim, :]
    w1c = em_W1[1 + hdim:1 + 2 * hdim, :]
    w1e = em_W1[1 + 2 * hdim:, :]

    a_tab, b_tab = _build_tables(x, h, w1r, w1c, em_b1.reshape(1, -1))
    p = _sc_gather(a_tab, b_tab, row, col)
    msg, faux = _edge_mlp(
        p, edge_fea, w1e, w1s,
        em_W2, em_b2.reshape(1, -1),
        co_W1, co_b1.reshape(1, -1), co_W2, co_b2.reshape(1, 1))
    mp, fp = _sc_scatter(row, msg, faux, n)
    x_new, h_new = _node_mlp(
        x, h, mp[:n], fp[:n],
        nn_W1[:hdim], nn_W1[hdim:], nn_b1.reshape(1, -1),
        nn_W2, nn_b2.reshape(1, -1))
    return (x_new, h_new)


# speedup vs baseline: 3.9482x; 1.0002x over previous
import functools

import jax
import jax.numpy as jnp
from jax import lax
from jax.experimental import pallas as pl
from jax.experimental.pallas import tpu as pltpu
from jax.experimental.pallas import tpu_sc as plsc

_NC = 2
_NS = 16
_NW = _NC * _NS
_CH = 80
_PW = 16
_L = 16
_CHS = 40


def _silu(v):
    return v * jax.nn.sigmoid(v)


def _tables_body(x_ref, h_ref, w1r_ref, w1c_ref, b1_ref, a_ref, b_ref):
    hb = h_ref[...]
    xb = x_ref[...]
    xpad = jnp.concatenate(
        [xb, jnp.zeros((xb.shape[0], 128 - xb.shape[1]), jnp.float32)], axis=1)
    a_ref[:, 0:128] = jnp.dot(hb, w1r_ref[...], preferred_element_type=jnp.float32) + b1_ref[...]
    a_ref[:, 128:256] = xpad
    b_ref[:, 0:128] = jnp.dot(hb, w1c_ref[...], preferred_element_type=jnp.float32)
    b_ref[:, 128:256] = -xpad


def _build_tables(x, h, w1r, w1c, b1):
    n, hdim = h.shape
    bn = 1000
    assert n % bn == 0
    return pl.pallas_call(
        _tables_body,
        grid=(n // bn,),
        in_specs=[
            pl.BlockSpec((bn, x.shape[1]), lambda i: (i, 0)),
            pl.BlockSpec((bn, hdim), lambda i: (i, 0)),
            pl.BlockSpec(w1r.shape, lambda i: (0, 0)),
            pl.BlockSpec(w1c.shape, lambda i: (0, 0)),
            pl.BlockSpec(b1.shape, lambda i: (0, 0)),
        ],
        out_specs=[pl.BlockSpec((bn, 256), lambda i: (i, 0))] * 2,
        out_shape=[jax.ShapeDtypeStruct((n, 256), jnp.float32)] * 2,
    )(x, h, w1r, w1c, b1)


def _sc_gather(a_tab, b_tab, row, col):
    e = row.shape[0]
    w = 128 + _PW
    assert e % (_NW * _CH) == 0
    ew = e // _NW
    n_ch = ew // _CH
    assert n_ch % 2 == 1 and n_ch >= 3
    mesh = plsc.VectorSubcoreMesh(
        core_axis_name="c", subcore_axis_name="s",
        num_cores=_NC, num_subcores=_NS)

    @functools.partial(
        pl.kernel,
        out_type=jax.ShapeDtypeStruct((e, w), jnp.float32),
        mesh=mesh,
        scratch_types=[
            pltpu.VMEM((_CH,), jnp.int32), pltpu.VMEM((_CH,), jnp.int32),
            pltpu.VMEM((_CH,), jnp.int32), pltpu.VMEM((_CH,), jnp.int32),
            pltpu.VMEM((_CH, 256), jnp.float32), pltpu.VMEM((_CH, 256), jnp.float32),
            pltpu.VMEM((_CH, 256), jnp.float32), pltpu.VMEM((_CH, 256), jnp.float32),
            pltpu.VMEM((_CH, w), jnp.float32), pltpu.VMEM((_CH, w), jnp.float32),
            pltpu.SemaphoreType.DMA, pltpu.SemaphoreType.DMA,
            pltpu.SemaphoreType.DMA, pltpu.SemaphoreType.DMA,
            pltpu.SemaphoreType.DMA, pltpu.SemaphoreType.DMA,
        ],
    )
    def gather_kernel(a_hbm, b_hbm, row_hbm, col_hbm, p_hbm,
                      ridx0, cidx0, ridx1, cidx1,
                      a0, b0, a1, b1, pb0, pb1,
                      si0, si1, sg0, sg1, so0, so1):
        wid = lax.axis_index("s") * _NC + lax.axis_index("c")
        base = wid * ew
        ridx = (ridx0, ridx1)
        cidx = (cidx0, cidx1)
        ab = ((a0, b0), (a1, b1))
        pb = (pb0, pb1)
        si = (si0, si1)
        sg = (sg0, sg1)
        so = (so0, so1)

        def start_idx(slot, ci):
            e0 = base + ci * _CH
            pltpu.async_copy(row_hbm.at[pl.ds(e0, _CH)], ridx[slot], si[slot])
            pltpu.async_copy(col_hbm.at[pl.ds(e0, _CH)], cidx[slot], si[slot])

        def wait_idx(slot):
            pltpu.make_async_copy(row_hbm.at[pl.ds(0, _CH)], ridx[slot], si[slot]).wait()
            pltpu.make_async_copy(col_hbm.at[pl.ds(0, _CH)], cidx[slot], si[slot]).wait()

        def start_gather(slot):
            pltpu.async_copy(a_hbm.at[ridx[slot]], ab[slot][0], sg[slot])
            pltpu.async_copy(b_hbm.at[cidx[slot]], ab[slot][1], sg[slot])

        def wait_gather(slot):
            pltpu.make_async_copy(a_hbm.at[ridx[slot]], ab[slot][0], sg[slot]).wait()
            pltpu.make_async_copy(b_hbm.at[cidx[slot]], ab[slot][1], sg[slot]).wait()

        def vadd(slot):
            b1_, b2_ = ab[slot]
            pbuf = pb[slot]

            def sum_row(r, c2):
                for cc in range(w // _L):
                    sl = pl.ds(cc * _L, _L)
                    pbuf[r, sl] = b1_[r, sl] + b2_[r, sl]
                return c2

            lax.fori_loop(0, _CH, sum_row, 0)

        def start_out(slot, ci):
            e0 = base + ci * _CH
            pltpu.async_copy(pb[slot], p_hbm.at[pl.ds(e0, _CH)], so[slot])

        def wait_out(slot):
            pltpu.make_async_copy(pb[slot], p_hbm.at[pl.ds(0, _CH)], so[slot]).wait()

        start_idx(0, 0)
        start_idx(1, 1)
        wait_idx(0)
        start_gather(0)

        def body(k, carry):
            c0 = 2 * k
            wait_idx(1)
            start_gather(1)
            wait_gather(0)
            start_idx(0, c0 + 2)

            @pl.when(k > 0)
            def _():
                wait_out(0)

            vadd(0)
            start_out(0, c0)
            wait_idx(0)
            start_gather(0)
            wait_gather(1)

            @pl.when(k < (n_ch - 3) // 2)
            def _():
                start_idx(1, c0 + 3)

            @pl.when(k > 0)
            def _():
                wait_out(1)

            vadd(1)
            start_out(1, c0 + 1)
            return carry

        lax.fori_loop(0, (n_ch - 1) // 2, body, 0)
        wait_gather(0)
        wait_out(0)
        vadd(0)
        start_out(0, n_ch - 1)
        wait_out(0)
        wait_out(1)

    return gather_kernel(a_tab, b_tab, row, col)


def _bdot(a, b):
    return jnp.dot(a.astype(jnp.bfloat16), b.astype(jnp.bfloat16),
                   preferred_element_type=jnp.float32)


def _edge_body(p_ref, ef_ref, w1e_ref, w1s_ref, w2_ref, b2_ref,
               cw1_ref, cb1_ref, cw2_ref, cb2_ref, msg_ref, faux_ref):
    p = p_ref[...]
    rij = p[:, 128:144]
    scal = jnp.sum(rij * rij, axis=1, keepdims=True)
    pre1 = (p[:, 0:128]
            + jnp.dot(ef_ref[...], w1e_ref[...], preferred_element_type=jnp.float32)
            + scal * w1s_ref[...])
    u = _silu(pre1)
    msg = _silu(_bdot(u, w2_ref[...]) + b2_ref[...])
    t = _silu(_bdot(msg, cw1_ref[...]) + cb1_ref[...])
    cm = _bdot(t, cw2_ref[...]) + cb2_ref[...]
    lane = lax.broadcasted_iota(jnp.int32, rij.shape, 1)
    faux = jnp.where(lane == 3, 1.0, rij * cm)
    msg_ref[...] = msg
    faux_ref[...] = faux


def _edge_mlp(p, edge_fea, w1e, w1s, w2, b2, cw1, cb1, cw2, cb2):
    e, w = p.shape
    be = 512
    assert e % be == 0
    full = lambda a: pl.BlockSpec(a.shape, lambda i: (0,) * a.ndim)
    return pl.pallas_call(
        _edge_body,
        grid=(e // be,),
        in_specs=[
            pl.BlockSpec((be, w), lambda i: (i, 0)),
            pl.BlockSpec((be, edge_fea.shape[1]), lambda i: (i, 0)),
            full(w1e), full(w1s), full(w2), full(b2),
            full(cw1), full(cb1), full(cw2), full(cb2),
        ],
        out_specs=[
            pl.BlockSpec((be, 128), lambda i: (i, 0)),
            pl.BlockSpec((be, _PW), lambda i: (i, 0)),
        ],
        out_shape=[
            jax.ShapeDtypeStruct((e, 128), jnp.float32),
            jax.ShapeDtypeStruct((e, _PW), jnp.float32),
        ],
    )(p, edge_fea, w1e, w1s, w2, b2, cw1, cb1, cw2, cb2)


def _sc_scatter(row, msg, faux, n):
    e = row.shape[0]
    npad = ((n + _NS * _CHS - 1) // (_NS * _CHS)) * (_NS * _CHS)
    ept = e // _NS
    assert e % (_NS * _CHS) == 0
    n_ch = ept // _CHS
    assert n_ch % 2 == 0
    rpt = npad // _NS
    assert rpt % _CHS == 0
    nzc = rpt // _CHS
    mesh = plsc.VectorSubcoreMesh(
        core_axis_name="c", subcore_axis_name="s",
        num_cores=_NC, num_subcores=_NS)

    @functools.partial(
        pl.kernel,
        out_type=[
            jax.ShapeDtypeStruct((npad, 128), jnp.float32),
            jax.ShapeDtypeStruct((npad, 128), jnp.float32),
        ],
        mesh=mesh,
        scratch_types=[
            pltpu.VMEM((_CHS,), jnp.int32), pltpu.VMEM((_CHS,), jnp.int32),
            pltpu.VMEM((_CHS, 128), jnp.float32), pltpu.VMEM((_CHS, 128), jnp.float32),
            pltpu.VMEM((_CHS, _PW), jnp.float32), pltpu.VMEM((_CHS, _PW), jnp.float32),
            pltpu.VMEM((_CHS, 128), jnp.float32), pltpu.VMEM((_CHS, 128), jnp.float32),
            pltpu.VMEM_SHARED((npad, 128), jnp.float32),
            pltpu.SemaphoreType.DMA, pltpu.SemaphoreType.DMA,
            pltpu.SemaphoreType.DMA, pltpu.SemaphoreType.DMA,
        ],
    )
    def scatter_kernel(row_hbm, msg_hbm, faux_hbm, mp_hbm, fp_hbm,
                       idx0, idx1, mbuf0, mbuf1, fbuf0, fbuf1, f0, f1, acc,
                       sl0, sl1, ss0, ss1):
        c = lax.axis_index("c")
        s = lax.axis_index("s")
        r0 = s * rpt
        base = s * ept
        zero16 = jnp.zeros((_L,), jnp.float32)
        idx = (idx0, idx1)
        mbuf = (mbuf0, mbuf1)
        fbuf = (fbuf0, fbuf1)
        f128 = (f0, f1)
        sl = (sl0, sl1)
        ss = (ss0, ss1)

        def zrow(r, c2):
            for cc in range(128 // _L):
                f0[r, pl.ds(cc * _L, _L)] = zero16
                f1[r, pl.ds(cc * _L, _L)] = zero16
            return c2

        lax.fori_loop(0, _CHS, zrow, 0)
        for k in range(nzc):
            pltpu.sync_copy(f0, acc.at[pl.ds(r0 + k * _CHS, _CHS)])
        plsc.subcore_barrier()

        def start_load(slot, ci, payload_hbm, payload_buf):
            e0 = base + ci * _CHS
            pltpu.async_copy(row_hbm.at[pl.ds(e0, _CHS)], idx[slot], sl[slot])
            pltpu.async_copy(payload_hbm.at[pl.ds(e0, _CHS)], payload_buf[slot], sl[slot])

        def wait_load(slot, payload_hbm, payload_buf):
            pltpu.make_async_copy(row_hbm.at[pl.ds(0, _CHS)], idx[slot], sl[slot]).wait()
            pltpu.make_async_copy(payload_hbm.at[pl.ds(0, _CHS)], payload_buf[slot], sl[slot]).wait()

        def start_scat(slot, payload_buf):
            pltpu.async_copy(payload_buf[slot], acc.at[idx[slot]], ss[slot], add=True)

        def wait_scat(slot, payload_buf):
            pltpu.make_async_copy(payload_buf[slot], acc.at[idx[slot]], ss[slot]).wait()

        @pl.when(c == 0)
        def _msg_loop():
            start_load(0, 0, msg_hbm, mbuf)
            start_load(1, 1, msg_hbm, mbuf)

            def body(k, carry):
                c0 = 2 * k
                wait_load(0, msg_hbm, mbuf)
                start_scat(0, mbuf)
                wait_load(1, msg_hbm, mbuf)
                start_scat(1, mbuf)
                wait_scat(0, mbuf)

                @pl.when(k < n_ch // 2 - 1)
                def _():
                    start_load(0, c0 + 2, msg_hbm, mbuf)

                wait_scat(1, mbuf)

                @pl.when(k < n_ch // 2 - 1)
                def _():
                    start_load(1, c0 + 3, msg_hbm, mbuf)

                return carry

            lax.fori_loop(0, n_ch // 2, body, 0)

        @pl.when(c == 1)
        def _faux_loop():
            start_load(0, 0, faux_hbm, fbuf)
            start_load(1, 1, faux_hbm, fbuf)

            def fcopy(slot):
                fb = fbuf[slot]
                fw = f128[slot]

                def frow(r, c2):
                    fw[r, pl.ds(0, _PW)] = fb[r, :]
                    return c2

                lax.fori_loop(0, _CHS, frow, 0)

            def body(k, carry):
                c0 = 2 * k
                wait_load(0, faux_hbm, fbuf)
                fcopy(0)
                start_scat(0, f128)
                wait_load(1, faux_hbm, fbuf)
                fcopy(1)
                start_scat(1, f128)
                wait_scat(0, f128)

                @pl.when(k < n_ch // 2 - 1)
                def _():
                    start_load(0, c0 + 2, faux_hbm, fbuf)

                wait_scat(1, f128)

                @pl.when(k < n_ch // 2 - 1)
                def _():
                    start_load(1, c0 + 3, faux_hbm, fbuf)

                return carry

            lax.fori_loop(0, n_ch // 2, body, 0)

        plsc.subcore_barrier()

        for k in range(nzc):
            pltpu.sync_copy(acc.at[pl.ds(r0 + k * _CHS, _CHS)], mbuf0)

            @pl.when(c == 0)
            def _wm():
                pltpu.sync_copy(mbuf0, mp_hbm.at[pl.ds(r0 + k * _CHS, _CHS)])

            @pl.when(c == 1)
            def _wf():
                pltpu.sync_copy(mbuf0, fp_hbm.at[pl.ds(r0 + k * _CHS, _CHS)])

    return scatter_kernel(row, msg, faux)


def _node_body(x_ref, h_ref, mp0_ref, fp0_ref,
               w1a_ref, w1b_ref, b1_ref, w2_ref, b2_ref, xo_ref, ho_ref):
    tm = mp0_ref[...]
    tf16 = fp0_ref[...]
    deg = tf16[:, 3:4]
    tf = jnp.clip(tf16[:, 0:3] / jnp.maximum(deg, 1.0), -100.0, 100.0)
    xo_ref[...] = x_ref[...] + tf
    z = _silu(jnp.dot(h_ref[...], w1a_ref[...], preferred_element_type=jnp.float32)
              + jnp.dot(tm, w1b_ref[...], preferred_element_type=jnp.float32)
              + b1_ref[...])
    ho_ref[...] = jnp.dot(z, w2_ref[...], preferred_element_type=jnp.float32) + b2_ref[...]


def _node_mlp(x, h, mp0, fp0, w1a, w1b, b1, w2, b2):
    n, hdim = h.shape
    bn = 1000
    assert n % bn == 0
    full = lambda a: pl.BlockSpec(a.shape, lambda i: (0,) * a.ndim)
    return pl.pallas_call(
        _node_body,
        grid=(n // bn,),
        in_specs=[
            pl.BlockSpec((bn, x.shape[1]), lambda i: (i, 0)),
            pl.BlockSpec((bn, hdim), lambda i: (i, 0)),
            pl.BlockSpec((bn, 128), lambda i: (i, 0)),
            pl.BlockSpec((bn, 128), lambda i: (i, 0)),
            full(w1a), full(w1b), full(b1), full(w2), full(b2),
        ],
        out_specs=[
            pl.BlockSpec((bn, x.shape[1]), lambda i: (i, 0)),
            pl.BlockSpec((bn, hdim), lambda i: (i, 0)),
        ],
        out_shape=[
            jax.ShapeDtypeStruct((n, x.shape[1]), jnp.float32),
            jax.ShapeDtypeStruct((n, hdim), jnp.float32),
        ],
    )(x, h, mp0, fp0, w1a, w1b, b1, w2, b2)


def kernel(x, h, edge_index, edge_fea,
           em_W1, em_b1, em_W2, em_b2,
           co_W1, co_b1, co_W2, co_b2,
           nn_W1, nn_b1, nn_W2, nn_b2):
    n, hdim = h.shape
    row = edge_index[0].astype(jnp.int32)
    col = edge_index[1].astype(jnp.int32)

    w1s = em_W1[0:1, :]
    w1r = em_W1[1:1 + hdim, :]
    w1c = em_W1[1 + hdim:1 + 2 * hdim, :]
    w1e = em_W1[1 + 2 * hdim:, :]

    a_tab, b_tab = _build_tables(x, h, w1r, w1c, em_b1.reshape(1, -1))
    p = _sc_gather(a_tab, b_tab, row, col)
    msg, faux = _edge_mlp(
        p, edge_fea, w1e, w1s,
        em_W2, em_b2.reshape(1, -1),
        co_W1, co_b1.reshape(1, -1), co_W2, co_b2.reshape(1, 1))
    mp, fp = _sc_scatter(row, msg, faux, n)
    x_new, h_new = _node_mlp(
        x, h, mp[:n], fp[:n],
        nn_W1[:hdim], nn_W1[hdim:], nn_b1.reshape(1, -1),
        nn_W2, nn_b2.reshape(1, -1))
    return (x_new, h_new)


# speedup vs baseline: 5.6207x; 1.4236x over previous
import functools

import jax
import jax.numpy as jnp
from jax import lax
from jax.experimental import pallas as pl
from jax.experimental.pallas import tpu as pltpu
from jax.experimental.pallas import tpu_sc as plsc

_NC = 2
_NS = 16
_NW = _NC * _NS
_CH = 80
_PW = 16
_L = 16
_CHS = 40


def _silu(v):
    return v * jax.nn.sigmoid(v)


def _tables_body(x_ref, h_ref, w1r_ref, w1c_ref, b1_ref, a_ref, b_ref):
    hb = h_ref[...]
    xb = x_ref[...]
    xpad = jnp.concatenate(
        [xb, jnp.zeros((xb.shape[0], 128 - xb.shape[1]), jnp.float32)], axis=1)
    a_ref[:, 0:128] = jnp.dot(hb, w1r_ref[...], preferred_element_type=jnp.float32) + b1_ref[...]
    a_ref[:, 128:256] = xpad
    b_ref[:, 0:128] = jnp.dot(hb, w1c_ref[...], preferred_element_type=jnp.float32)
    b_ref[:, 128:256] = -xpad


def _build_tables(x, h, w1r, w1c, b1):
    n, hdim = h.shape
    bn = 1000
    assert n % bn == 0
    return pl.pallas_call(
        _tables_body,
        grid=(n // bn,),
        in_specs=[
            pl.BlockSpec((bn, x.shape[1]), lambda i: (i, 0)),
            pl.BlockSpec((bn, hdim), lambda i: (i, 0)),
            pl.BlockSpec(w1r.shape, lambda i: (0, 0)),
            pl.BlockSpec(w1c.shape, lambda i: (0, 0)),
            pl.BlockSpec(b1.shape, lambda i: (0, 0)),
        ],
        out_specs=[pl.BlockSpec((bn, 256), lambda i: (i, 0))] * 2,
        out_shape=[jax.ShapeDtypeStruct((n, 256), jnp.float32)] * 2,
    )(x, h, w1r, w1c, b1)


def _sc_gather(a_tab, b_tab, row, col, ch):
    e = row.shape[0]
    w = 128 + _PW
    assert e % (_NW * ch) == 0
    ew = e // _NW
    n_ch = ew // ch
    assert n_ch % 2 == 1 and n_ch >= 3
    mesh = plsc.VectorSubcoreMesh(
        core_axis_name="c", subcore_axis_name="s",
        num_cores=_NC, num_subcores=_NS)

    @functools.partial(
        pl.kernel,
        out_type=jax.ShapeDtypeStruct((e, w), jnp.float32),
        mesh=mesh,
        scratch_types=[
            pltpu.VMEM((ch,), jnp.int32), pltpu.VMEM((ch,), jnp.int32),
            pltpu.VMEM((ch,), jnp.int32), pltpu.VMEM((ch,), jnp.int32),
            pltpu.VMEM((ch, 256), jnp.float32), pltpu.VMEM((ch, 256), jnp.float32),
            pltpu.VMEM((ch, 256), jnp.float32), pltpu.VMEM((ch, 256), jnp.float32),
            pltpu.VMEM((ch, w), jnp.float32), pltpu.VMEM((ch, w), jnp.float32),
            pltpu.SemaphoreType.DMA, pltpu.SemaphoreType.DMA,
            pltpu.SemaphoreType.DMA, pltpu.SemaphoreType.DMA,
            pltpu.SemaphoreType.DMA, pltpu.SemaphoreType.DMA,
        ],
    )
    def gather_kernel(a_hbm, b_hbm, row_hbm, col_hbm, p_hbm,
                      ridx0, cidx0, ridx1, cidx1,
                      a0, b0, a1, b1, pb0, pb1,
                      si0, si1, sg0, sg1, so0, so1):
        wid = lax.axis_index("s") * _NC + lax.axis_index("c")
        base = wid * ew
        ridx = (ridx0, ridx1)
        cidx = (cidx0, cidx1)
        ab = ((a0, b0), (a1, b1))
        pb = (pb0, pb1)
        si = (si0, si1)
        sg = (sg0, sg1)
        so = (so0, so1)

        def start_idx(slot, ci):
            e0 = base + ci * ch
            pltpu.async_copy(row_hbm.at[pl.ds(e0, ch)], ridx[slot], si[slot])
            pltpu.async_copy(col_hbm.at[pl.ds(e0, ch)], cidx[slot], si[slot])

        def wait_idx(slot):
            pltpu.make_async_copy(row_hbm.at[pl.ds(0, ch)], ridx[slot], si[slot]).wait()
            pltpu.make_async_copy(col_hbm.at[pl.ds(0, ch)], cidx[slot], si[slot]).wait()

        def start_gather(slot):
            pltpu.async_copy(a_hbm.at[ridx[slot]], ab[slot][0], sg[slot])
            pltpu.async_copy(b_hbm.at[cidx[slot]], ab[slot][1], sg[slot])

        def wait_gather(slot):
            pltpu.make_async_copy(a_hbm.at[ridx[slot]], ab[slot][0], sg[slot]).wait()
            pltpu.make_async_copy(b_hbm.at[cidx[slot]], ab[slot][1], sg[slot]).wait()

        def vadd(slot):
            b1_, b2_ = ab[slot]
            pbuf = pb[slot]

            def sum_row(r, c2):
                for cc in range(w // _L):
                    sl = pl.ds(cc * _L, _L)
                    pbuf[r, sl] = b1_[r, sl] + b2_[r, sl]
                return c2

            lax.fori_loop(0, ch, sum_row, 0)

        def start_out(slot, ci):
            e0 = base + ci * ch
            pltpu.async_copy(pb[slot], p_hbm.at[pl.ds(e0, ch)], so[slot])

        def wait_out(slot):
            pltpu.make_async_copy(pb[slot], p_hbm.at[pl.ds(0, ch)], so[slot]).wait()

        start_idx(0, 0)
        start_idx(1, 1)
        wait_idx(0)
        start_gather(0)

        def body(k, carry):
            c0 = 2 * k
            wait_idx(1)
            start_gather(1)
            wait_gather(0)
            start_idx(0, c0 + 2)

            @pl.when(k > 0)
            def _():
                wait_out(0)

            vadd(0)
            start_out(0, c0)
            wait_idx(0)
            start_gather(0)
            wait_gather(1)

            @pl.when(k < (n_ch - 3) // 2)
            def _():
                start_idx(1, c0 + 3)

            @pl.when(k > 0)
            def _():
                wait_out(1)

            vadd(1)
            start_out(1, c0 + 1)
            return carry

        lax.fori_loop(0, (n_ch - 1) // 2, body, 0)
        wait_gather(0)
        wait_out(0)
        vadd(0)
        start_out(0, n_ch - 1)
        wait_out(0)
        wait_out(1)

    return gather_kernel(a_tab, b_tab, row, col)


def _bdot(a, b):
    return jnp.dot(a.astype(jnp.bfloat16), b.astype(jnp.bfloat16),
                   preferred_element_type=jnp.float32)


def _edge_body(p_ref, ef_ref, w1e_ref, w1s_ref, w2_ref, b2_ref,
               cw1_ref, cb1_ref, cw2_ref, cb2_ref, msg_ref, faux_ref):
    p = p_ref[...]
    rij = p[:, 128:144]
    scal = jnp.sum(rij * rij, axis=1, keepdims=True)
    pre1 = (p[:, 0:128]
            + jnp.dot(ef_ref[...], w1e_ref[...], preferred_element_type=jnp.float32)
            + scal * w1s_ref[...])
    u = _silu(pre1)
    msg = _silu(_bdot(u, w2_ref[...]) + b2_ref[...])
    t = _silu(_bdot(msg, cw1_ref[...]) + cb1_ref[...])
    cm = _bdot(t, cw2_ref[...]) + cb2_ref[...]
    lane = lax.broadcasted_iota(jnp.int32, rij.shape, 1)
    faux = jnp.where(lane == 3, 1.0, rij * cm)
    msg_ref[...] = msg
    faux_ref[...] = faux


def _edge_mlp(p, edge_fea, w1e, w1s, w2, b2, cw1, cb1, cw2, cb2):
    e, w = p.shape
    be = 1280
    assert e % be == 0
    full = lambda a: pl.BlockSpec(a.shape, lambda i: (0,) * a.ndim)
    return pl.pallas_call(
        _edge_body,
        grid=(e // be,),
        in_specs=[
            pl.BlockSpec((be, w), lambda i: (i, 0)),
            pl.BlockSpec((be, edge_fea.shape[1]), lambda i: (i, 0)),
            full(w1e), full(w1s), full(w2), full(b2),
            full(cw1), full(cb1), full(cw2), full(cb2),
        ],
        out_specs=[
            pl.BlockSpec((be, 128), lambda i: (i, 0)),
            pl.BlockSpec((be, _PW), lambda i: (i, 0)),
        ],
        out_shape=[
            jax.ShapeDtypeStruct((e, 128), jnp.float32),
            jax.ShapeDtypeStruct((e, _PW), jnp.float32),
        ],
    )(p, edge_fea, w1e, w1s, w2, b2, cw1, cb1, cw2, cb2)


def _sc_scatter(row, msg, faux, n):
    e = row.shape[0]
    npad = ((n + _NS * _CHS - 1) // (_NS * _CHS)) * (_NS * _CHS)
    ept = e // _NS
    assert e % (_NS * _CHS) == 0
    n_ch = ept // _CHS
    assert n_ch % 2 == 0
    rpt = npad // _NS
    assert rpt % _CHS == 0
    nzc = rpt // _CHS
    mesh = plsc.VectorSubcoreMesh(
        core_axis_name="c", subcore_axis_name="s",
        num_cores=_NC, num_subcores=_NS)

    @functools.partial(
        pl.kernel,
        out_type=[
            jax.ShapeDtypeStruct((npad, 128), jnp.float32),
            jax.ShapeDtypeStruct((npad, 128), jnp.float32),
        ],
        mesh=mesh,
        scratch_types=[
            pltpu.VMEM((_CHS,), jnp.int32), pltpu.VMEM((_CHS,), jnp.int32),
            pltpu.VMEM((_CHS, 128), jnp.float32), pltpu.VMEM((_CHS, 128), jnp.float32),
            pltpu.VMEM((_CHS, _PW), jnp.float32), pltpu.VMEM((_CHS, _PW), jnp.float32),
            pltpu.VMEM((_CHS, 128), jnp.float32), pltpu.VMEM((_CHS, 128), jnp.float32),
            pltpu.VMEM_SHARED((npad, 128), jnp.float32),
            pltpu.SemaphoreType.DMA, pltpu.SemaphoreType.DMA,
            pltpu.SemaphoreType.DMA, pltpu.SemaphoreType.DMA,
        ],
    )
    def scatter_kernel(row_hbm, msg_hbm, faux_hbm, mp_hbm, fp_hbm,
                       idx0, idx1, mbuf0, mbuf1, fbuf0, fbuf1, f0, f1, acc,
                       sl0, sl1, ss0, ss1):
        c = lax.axis_index("c")
        s = lax.axis_index("s")
        r0 = s * rpt
        base = s * ept
        zero16 = jnp.zeros((_L,), jnp.float32)
        idx = (idx0, idx1)
        mbuf = (mbuf0, mbuf1)
        fbuf = (fbuf0, fbuf1)
        f128 = (f0, f1)
        sl = (sl0, sl1)
        ss = (ss0, ss1)

        def zrow(r, c2):
            for cc in range(128 // _L):
                f0[r, pl.ds(cc * _L, _L)] = zero16
                f1[r, pl.ds(cc * _L, _L)] = zero16
            return c2

        lax.fori_loop(0, _CHS, zrow, 0)
        for k in range(nzc):
            pltpu.sync_copy(f0, acc.at[pl.ds(r0 + k * _CHS, _CHS)])
        plsc.subcore_barrier()

        def start_load(slot, ci, payload_hbm, payload_buf):
            e0 = base + ci * _CHS
            pltpu.async_copy(row_hbm.at[pl.ds(e0, _CHS)], idx[slot], sl[slot])
            pltpu.async_copy(payload_hbm.at[pl.ds(e0, _CHS)], payload_buf[slot], sl[slot])

        def wait_load(slot, payload_hbm, payload_buf):
            pltpu.make_async_copy(row_hbm.at[pl.ds(0, _CHS)], idx[slot], sl[slot]).wait()
            pltpu.make_async_copy(payload_hbm.at[pl.ds(0, _CHS)], payload_buf[slot], sl[slot]).wait()

        def start_scat(slot, payload_buf):
            pltpu.async_copy(payload_buf[slot], acc.at[idx[slot]], ss[slot], add=True)

        def wait_scat(slot, payload_buf):
            pltpu.make_async_copy(payload_buf[slot], acc.at[idx[slot]], ss[slot]).wait()

        @pl.when(c == 0)
        def _msg_loop():
            start_load(0, 0, msg_hbm, mbuf)
            start_load(1, 1, msg_hbm, mbuf)

            def body(k, carry):
                c0 = 2 * k
                wait_load(0, msg_hbm, mbuf)
                start_scat(0, mbuf)
                wait_load(1, msg_hbm, mbuf)
                start_scat(1, mbuf)
                wait_scat(0, mbuf)

                @pl.when(k < n_ch // 2 - 1)
                def _():
                    start_load(0, c0 + 2, msg_hbm, mbuf)

                wait_scat(1, mbuf)

                @pl.when(k < n_ch // 2 - 1)
                def _():
                    start_load(1, c0 + 3, msg_hbm, mbuf)

                return carry

            lax.fori_loop(0, n_ch // 2, body, 0)

        @pl.when(c == 1)
        def _faux_loop():
            start_load(0, 0, faux_hbm, fbuf)
            start_load(1, 1, faux_hbm, fbuf)

            def fcopy(slot):
                fb = fbuf[slot]
                fw = f128[slot]

                def frow(r, c2):
                    fw[r, pl.ds(0, _PW)] = fb[r, :]
                    return c2

                lax.fori_loop(0, _CHS, frow, 0)

            def body(k, carry):
                c0 = 2 * k
                wait_load(0, faux_hbm, fbuf)
                fcopy(0)
                start_scat(0, f128)
                wait_load(1, faux_hbm, fbuf)
                fcopy(1)
                start_scat(1, f128)
                wait_scat(0, f128)

                @pl.when(k < n_ch // 2 - 1)
                def _():
                    start_load(0, c0 + 2, faux_hbm, fbuf)

                wait_scat(1, f128)

                @pl.when(k < n_ch // 2 - 1)
                def _():
                    start_load(1, c0 + 3, faux_hbm, fbuf)

                return carry

            lax.fori_loop(0, n_ch // 2, body, 0)

        plsc.subcore_barrier()

        for k in range(nzc):
            pltpu.sync_copy(acc.at[pl.ds(r0 + k * _CHS, _CHS)], mbuf0)

            @pl.when(c == 0)
            def _wm():
                pltpu.sync_copy(mbuf0, mp_hbm.at[pl.ds(r0 + k * _CHS, _CHS)])

            @pl.when(c == 1)
            def _wf():
                pltpu.sync_copy(mbuf0, fp_hbm.at[pl.ds(r0 + k * _CHS, _CHS)])

    return scatter_kernel(row, msg, faux)


def _node_body(x_ref, h_ref, mp0_ref, mp1_ref, fp0_ref, fp1_ref,
               w1a_ref, w1b_ref, b1_ref, w2_ref, b2_ref, xo_ref, ho_ref):
    tm = mp0_ref[...] + mp1_ref[...]
    tf16 = fp0_ref[...] + fp1_ref[...]
    deg = tf16[:, 3:4]
    tf = jnp.clip(tf16[:, 0:3] / jnp.maximum(deg, 1.0), -100.0, 100.0)
    xo_ref[...] = x_ref[...] + tf
    z = _silu(jnp.dot(h_ref[...], w1a_ref[...], preferred_element_type=jnp.float32)
              + jnp.dot(tm, w1b_ref[...], preferred_element_type=jnp.float32)
              + b1_ref[...])
    ho_ref[...] = jnp.dot(z, w2_ref[...], preferred_element_type=jnp.float32) + b2_ref[...]


def _node_mlp(x, h, mp0, mp1, fp0, fp1, w1a, w1b, b1, w2, b2):
    n, hdim = h.shape
    bn = 1000
    assert n % bn == 0
    full = lambda a: pl.BlockSpec(a.shape, lambda i: (0,) * a.ndim)
    return pl.pallas_call(
        _node_body,
        grid=(n // bn,),
        in_specs=[
            pl.BlockSpec((bn, x.shape[1]), lambda i: (i, 0)),
            pl.BlockSpec((bn, hdim), lambda i: (i, 0)),
            pl.BlockSpec((bn, 128), lambda i: (i, 0)),
            pl.BlockSpec((bn, 128), lambda i: (i, 0)),
            pl.BlockSpec((bn, 128), lambda i: (i, 0)),
            pl.BlockSpec((bn, 128), lambda i: (i, 0)),
            full(w1a), full(w1b), full(b1), full(w2), full(b2),
        ],
        out_specs=[
            pl.BlockSpec((bn, x.shape[1]), lambda i: (i, 0)),
            pl.BlockSpec((bn, hdim), lambda i: (i, 0)),
        ],
        out_shape=[
            jax.ShapeDtypeStruct((n, x.shape[1]), jnp.float32),
            jax.ShapeDtypeStruct((n, hdim), jnp.float32),
        ],
    )(x, h, mp0, mp1, fp0, fp1, w1a, w1b, b1, w2, b2)


def kernel(x, h, edge_index, edge_fea,
           em_W1, em_b1, em_W2, em_b2,
           co_W1, co_b1, co_W2, co_b2,
           nn_W1, nn_b1, nn_W2, nn_b2):
    n, hdim = h.shape
    row = edge_index[0].astype(jnp.int32)
    col = edge_index[1].astype(jnp.int32)

    w1s = em_W1[0:1, :]
    w1r = em_W1[1:1 + hdim, :]
    w1c = em_W1[1 + hdim:1 + 2 * hdim, :]
    w1e = em_W1[1 + 2 * hdim:, :]

    a_tab, b_tab = _build_tables(x, h, w1r, w1c, em_b1.reshape(1, -1))

    e = row.shape[0]
    eh = e // 2
    mps, fps = [], []
    edge_args = (w1e, w1s, em_W2, em_b2.reshape(1, -1),
                 co_W1, co_b1.reshape(1, -1), co_W2, co_b2.reshape(1, 1))
    for half in range(2):
        sl = slice(half * eh, (half + 1) * eh)
        p_h = _sc_gather(a_tab, b_tab, row[sl], col[sl], 40)
        msg_h, faux_h = _edge_mlp(p_h, edge_fea[sl], *edge_args)
        mp_h, fp_h = _sc_scatter(row[sl], msg_h, faux_h, n)
        mps.append(mp_h)
        fps.append(fp_h)
    x_new, h_new = _node_mlp(
        x, h, mps[0][:n], mps[1][:n], fps[0][:n], fps[1][:n],
        nn_W1[:hdim], nn_W1[hdim:], nn_b1.reshape(1, -1),
        nn_W2, nn_b2.reshape(1, -1))
    return (x_new, h_new)


# speedup vs baseline: 6.1308x; 1.0908x over previous
import functools

import jax
import jax.numpy as jnp
from jax import lax
from jax.experimental import pallas as pl
from jax.experimental.pallas import tpu as pltpu
from jax.experimental.pallas import tpu_sc as plsc

_NC = 2
_NS = 16
_NW = _NC * _NS
_CH = 80
_PW = 16
_L = 16
_CHS = 40


def _silu(v):
    return v * jax.nn.sigmoid(v)


def _pack_bf16(lo, hi):
    bits = lambda v: lax.bitcast_convert_type(
        v.astype(jnp.bfloat16).astype(jnp.float32), jnp.uint32)
    return lax.bitcast_convert_type(
        (bits(lo) >> 16) | (bits(hi) & jnp.uint32(0xFFFF0000)), jnp.float32)


def _tables_body(x_ref, h_ref, w1r_ref, w1c_ref, b1_ref, a_ref, b_ref):
    hb = h_ref[...]
    xb = x_ref[...]
    xpad = jnp.concatenate(
        [xb, jnp.zeros((xb.shape[0], 128 - xb.shape[1]), jnp.float32)], axis=1)
    ap = jnp.dot(hb, w1r_ref[...], preferred_element_type=jnp.float32) + b1_ref[...]
    bp = jnp.dot(hb, w1c_ref[...], preferred_element_type=jnp.float32)
    a_ref[...] = _pack_bf16(ap, xpad)
    b_ref[...] = _pack_bf16(bp, -xpad)


def _build_tables(x, h, w1r, w1c, b1):
    n, hdim = h.shape
    bn = 1000
    assert n % bn == 0
    return pl.pallas_call(
        _tables_body,
        grid=(n // bn,),
        in_specs=[
            pl.BlockSpec((bn, x.shape[1]), lambda i: (i, 0)),
            pl.BlockSpec((bn, hdim), lambda i: (i, 0)),
            pl.BlockSpec(w1r.shape, lambda i: (0, 0)),
            pl.BlockSpec(w1c.shape, lambda i: (0, 0)),
            pl.BlockSpec(b1.shape, lambda i: (0, 0)),
        ],
        out_specs=[pl.BlockSpec((bn, 128), lambda i: (i, 0))] * 2,
        out_shape=[jax.ShapeDtypeStruct((n, 128), jnp.float32)] * 2,
    )(x, h, w1r, w1c, b1)


def _sc_gather(a_tab, b_tab, row, col, ch):
    e = row.shape[0]
    assert e % (_NW * ch) == 0
    ew = e // _NW
    n_ch = ew // ch
    assert n_ch % 2 == 1 and n_ch >= 3
    mesh = plsc.VectorSubcoreMesh(
        core_axis_name="c", subcore_axis_name="s",
        num_cores=_NC, num_subcores=_NS)

    @functools.partial(
        pl.kernel,
        out_type=[jax.ShapeDtypeStruct((e, 128), jnp.float32)] * 2,
        mesh=mesh,
        scratch_types=[
            pltpu.VMEM((ch,), jnp.int32), pltpu.VMEM((ch,), jnp.int32),
            pltpu.VMEM((ch,), jnp.int32), pltpu.VMEM((ch,), jnp.int32),
            pltpu.VMEM((ch, 128), jnp.float32), pltpu.VMEM((ch, 128), jnp.float32),
            pltpu.VMEM((ch, 128), jnp.float32), pltpu.VMEM((ch, 128), jnp.float32),
            pltpu.SemaphoreType.DMA, pltpu.SemaphoreType.DMA,
            pltpu.SemaphoreType.DMA, pltpu.SemaphoreType.DMA,
            pltpu.SemaphoreType.DMA, pltpu.SemaphoreType.DMA,
        ],
    )
    def gather_kernel(a_hbm, b_hbm, row_hbm, col_hbm, g1_hbm, g2_hbm,
                      ridx0, cidx0, ridx1, cidx1,
                      a0, b0, a1, b1,
                      si0, si1, sg0, sg1, so0, so1):
        wid = lax.axis_index("s") * _NC + lax.axis_index("c")
        base = wid * ew
        ridx = (ridx0, ridx1)
        cidx = (cidx0, cidx1)
        ab = ((a0, b0), (a1, b1))
        si = (si0, si1)
        sg = (sg0, sg1)
        so = (so0, so1)

        def start_idx(slot, ci):
            e0 = base + ci * ch
            pltpu.async_copy(row_hbm.at[pl.ds(e0, ch)], ridx[slot], si[slot])
            pltpu.async_copy(col_hbm.at[pl.ds(e0, ch)], cidx[slot], si[slot])

        def wait_idx(slot):
            pltpu.make_async_copy(row_hbm.at[pl.ds(0, ch)], ridx[slot], si[slot]).wait()
            pltpu.make_async_copy(col_hbm.at[pl.ds(0, ch)], cidx[slot], si[slot]).wait()

        def start_gather(slot):
            pltpu.async_copy(a_hbm.at[ridx[slot]], ab[slot][0], sg[slot])
            pltpu.async_copy(b_hbm.at[cidx[slot]], ab[slot][1], sg[slot])

        def wait_gather(slot):
            pltpu.make_async_copy(a_hbm.at[ridx[slot]], ab[slot][0], sg[slot]).wait()
            pltpu.make_async_copy(b_hbm.at[cidx[slot]], ab[slot][1], sg[slot]).wait()

        def start_out(slot, ci):
            e0 = base + ci * ch
            pltpu.async_copy(ab[slot][0], g1_hbm.at[pl.ds(e0, ch)], so[slot])
            pltpu.async_copy(ab[slot][1], g2_hbm.at[pl.ds(e0, ch)], so[slot])

        def wait_out(slot):
            pltpu.make_async_copy(ab[slot][0], g1_hbm.at[pl.ds(0, ch)], so[slot]).wait()
            pltpu.make_async_copy(ab[slot][1], g2_hbm.at[pl.ds(0, ch)], so[slot]).wait()

        start_idx(0, 0)
        start_idx(1, 1)
        wait_idx(0)
        start_gather(0)

        def body(k, carry):
            c0 = 2 * k
            wait_idx(1)

            @pl.when(k > 0)
            def _():
                wait_out(1)

            start_gather(1)
            wait_gather(0)
            start_idx(0, c0 + 2)
            start_out(0, c0)
            wait_idx(0)
            wait_out(0)
            start_gather(0)
            wait_gather(1)

            @pl.when(k < (n_ch - 3) // 2)
            def _():
                start_idx(1, c0 + 3)

            start_out(1, c0 + 1)
            return carry

        lax.fori_loop(0, (n_ch - 1) // 2, body, 0)
        wait_gather(0)
        start_out(0, n_ch - 1)
        wait_out(0)
        wait_out(1)

    return gather_kernel(a_tab, b_tab, row, col)


def _bdot(a, b):
    return jnp.dot(a.astype(jnp.bfloat16), b.astype(jnp.bfloat16),
                   preferred_element_type=jnp.float32)


def _edge_body(g1_ref, g2_ref, ef_ref, w1e_ref, w1s_ref, w2_ref, b2_ref,
               cw1_ref, cb1_ref, cw2_ref, cb2_ref, msg_ref, faux_ref):
    him = jnp.uint32(0xFFFF0000)
    g1u = lax.bitcast_convert_type(g1_ref[...], jnp.uint32)
    g2u = lax.bitcast_convert_type(g2_ref[...], jnp.uint32)
    s1 = (lax.bitcast_convert_type(g1u << 16, jnp.float32)
          + lax.bitcast_convert_type(g2u << 16, jnp.float32))
    rijf = (lax.bitcast_convert_type(g1u & him, jnp.float32)
            + lax.bitcast_convert_type(g2u & him, jnp.float32))
    rij = rijf[:, 0:16]
    scal = jnp.sum(rijf * rijf, axis=1, keepdims=True)
    pre1 = (s1
            + jnp.dot(ef_ref[...], w1e_ref[...], preferred_element_type=jnp.float32)
            + scal * w1s_ref[...])
    u = _silu(pre1)
    msg = _silu(_bdot(u, w2_ref[...]) + b2_ref[...])
    t = _silu(_bdot(msg, cw1_ref[...]) + cb1_ref[...])
    cm = _bdot(t, cw2_ref[...]) + cb2_ref[...]
    lane = lax.broadcasted_iota(jnp.int32, rij.shape, 1)
    faux = jnp.where(lane == 3, 1.0, rij * cm)
    msg_ref[...] = msg
    faux_ref[...] = faux


def _edge_mlp(g1, g2, edge_fea, w1e, w1s, w2, b2, cw1, cb1, cw2, cb2):
    e, w = g1.shape
    assert w == 128
    be = 1280
    assert e % be == 0
    full = lambda a: pl.BlockSpec(a.shape, lambda i: (0,) * a.ndim)
    return pl.pallas_call(
        _edge_body,
        grid=(e // be,),
        in_specs=[
            pl.BlockSpec((be, w), lambda i: (i, 0)),
            pl.BlockSpec((be, w), lambda i: (i, 0)),
            pl.BlockSpec((be, edge_fea.shape[1]), lambda i: (i, 0)),
            full(w1e), full(w1s), full(w2), full(b2),
            full(cw1), full(cb1), full(cw2), full(cb2),
        ],
        out_specs=[
            pl.BlockSpec((be, 128), lambda i: (i, 0)),
            pl.BlockSpec((be, _PW), lambda i: (i, 0)),
        ],
        out_shape=[
            jax.ShapeDtypeStruct((e, 128), jnp.float32),
            jax.ShapeDtypeStruct((e, _PW), jnp.float32),
        ],
    )(g1, g2, edge_fea, w1e, w1s, w2, b2, cw1, cb1, cw2, cb2)


def _sc_scatter(row, msg, faux, n):
    e = row.shape[0]
    npad = ((n + _NS * _CHS - 1) // (_NS * _CHS)) * (_NS * _CHS)
    ept = e // _NS
    assert e % (_NS * _CHS) == 0
    n_ch = ept // _CHS
    assert n_ch % 2 == 0
    rpt = npad // _NS
    assert rpt % _CHS == 0
    nzc = rpt // _CHS
    mesh = plsc.VectorSubcoreMesh(
        core_axis_name="c", subcore_axis_name="s",
        num_cores=_NC, num_subcores=_NS)

    @functools.partial(
        pl.kernel,
        out_type=[
            jax.ShapeDtypeStruct((npad, 128), jnp.float32),
            jax.ShapeDtypeStruct((npad, 128), jnp.float32),
        ],
        mesh=mesh,
        scratch_types=[
            pltpu.VMEM((_CHS,), jnp.int32), pltpu.VMEM((_CHS,), jnp.int32),
            pltpu.VMEM((_CHS, 128), jnp.float32), pltpu.VMEM((_CHS, 128), jnp.float32),
            pltpu.VMEM((_CHS, _PW), jnp.float32), pltpu.VMEM((_CHS, _PW), jnp.float32),
            pltpu.VMEM((_CHS, 128), jnp.float32), pltpu.VMEM((_CHS, 128), jnp.float32),
            pltpu.VMEM_SHARED((npad, 128), jnp.float32),
            pltpu.SemaphoreType.DMA, pltpu.SemaphoreType.DMA,
            pltpu.SemaphoreType.DMA, pltpu.SemaphoreType.DMA,
        ],
    )
    def scatter_kernel(row_hbm, msg_hbm, faux_hbm, mp_hbm, fp_hbm,
                       idx0, idx1, mbuf0, mbuf1, fbuf0, fbuf1, f0, f1, acc,
                       sl0, sl1, ss0, ss1):
        c = lax.axis_index("c")
        s = lax.axis_index("s")
        r0 = s * rpt
        base = s * ept
        zero16 = jnp.zeros((_L,), jnp.float32)
        idx = (idx0, idx1)
        mbuf = (mbuf0, mbuf1)
        fbuf = (fbuf0, fbuf1)
        f128 = (f0, f1)
        sl = (sl0, sl1)
        ss = (ss0, ss1)

        def zrow(r, c2):
            for cc in range(128 // _L):
                f0[r, pl.ds(cc * _L, _L)] = zero16
                f1[r, pl.ds(cc * _L, _L)] = zero16
            return c2

        lax.fori_loop(0, _CHS, zrow, 0)
        for k in range(nzc):
            pltpu.sync_copy(f0, acc.at[pl.ds(r0 + k * _CHS, _CHS)])
        plsc.subcore_barrier()

        def start_load(slot, ci, payload_hbm, payload_buf):
            e0 = base + ci * _CHS
            pltpu.async_copy(row_hbm.at[pl.ds(e0, _CHS)], idx[slot], sl[slot])
            pltpu.async_copy(payload_hbm.at[pl.ds(e0, _CHS)], payload_buf[slot], sl[slot])

        def wait_load(slot, payload_hbm, payload_buf):
            pltpu.make_async_copy(row_hbm.at[pl.ds(0, _CHS)], idx[slot], sl[slot]).wait()
            pltpu.make_async_copy(payload_hbm.at[pl.ds(0, _CHS)], payload_buf[slot], sl[slot]).wait()

        def start_scat(slot, payload_buf):
            pltpu.async_copy(payload_buf[slot], acc.at[idx[slot]], ss[slot], add=True)

        def wait_scat(slot, payload_buf):
            pltpu.make_async_copy(payload_buf[slot], acc.at[idx[slot]], ss[slot]).wait()

        @pl.when(c == 0)
        def _msg_loop():
            start_load(0, 0, msg_hbm, mbuf)
            start_load(1, 1, msg_hbm, mbuf)

            def body(k, carry):
                c0 = 2 * k
                wait_load(0, msg_hbm, mbuf)
                start_scat(0, mbuf)
                wait_load(1, msg_hbm, mbuf)
                start_scat(1, mbuf)
                wait_scat(0, mbuf)

                @pl.when(k < n_ch // 2 - 1)
                def _():
                    start_load(0, c0 + 2, msg_hbm, mbuf)

                wait_scat(1, mbuf)

                @pl.when(k < n_ch // 2 - 1)
                def _():
                    start_load(1, c0 + 3, msg_hbm, mbuf)

                return carry

            lax.fori_loop(0, n_ch // 2, body, 0)

        @pl.when(c == 1)
        def _faux_loop():
            start_load(0, 0, faux_hbm, fbuf)
            start_load(1, 1, faux_hbm, fbuf)

            def fcopy(slot):
                fb = fbuf[slot]
                fw = f128[slot]

                def frow(r, c2):
                    fw[r, pl.ds(0, _PW)] = fb[r, :]
                    return c2

                lax.fori_loop(0, _CHS, frow, 0)

            def body(k, carry):
                c0 = 2 * k
                wait_load(0, faux_hbm, fbuf)
                fcopy(0)
                start_scat(0, f128)
                wait_load(1, faux_hbm, fbuf)
                fcopy(1)
                start_scat(1, f128)
                wait_scat(0, f128)

                @pl.when(k < n_ch // 2 - 1)
                def _():
                    start_load(0, c0 + 2, faux_hbm, fbuf)

                wait_scat(1, f128)

                @pl.when(k < n_ch // 2 - 1)
                def _():
                    start_load(1, c0 + 3, faux_hbm, fbuf)

                return carry

            lax.fori_loop(0, n_ch // 2, body, 0)

        plsc.subcore_barrier()

        for k in range(nzc):
            pltpu.sync_copy(acc.at[pl.ds(r0 + k * _CHS, _CHS)], mbuf0)

            @pl.when(c == 0)
            def _wm():
                pltpu.sync_copy(mbuf0, mp_hbm.at[pl.ds(r0 + k * _CHS, _CHS)])

            @pl.when(c == 1)
            def _wf():
                pltpu.sync_copy(mbuf0, fp_hbm.at[pl.ds(r0 + k * _CHS, _CHS)])

    return scatter_kernel(row, msg, faux)


def _node_body(x_ref, h_ref, mp0_ref, mp1_ref, fp0_ref, fp1_ref,
               w1a_ref, w1b_ref, b1_ref, w2_ref, b2_ref, xo_ref, ho_ref):
    tm = mp0_ref[...] + mp1_ref[...]
    tf16 = fp0_ref[...] + fp1_ref[...]
    deg = tf16[:, 3:4]
    tf = jnp.clip(tf16[:, 0:3] / jnp.maximum(deg, 1.0), -100.0, 100.0)
    xo_ref[...] = x_ref[...] + tf
    z = _silu(jnp.dot(h_ref[...], w1a_ref[...], preferred_element_type=jnp.float32)
              + jnp.dot(tm, w1b_ref[...], preferred_element_type=jnp.float32)
              + b1_ref[...])
    ho_ref[...] = jnp.dot(z, w2_ref[...], preferred_element_type=jnp.float32) + b2_ref[...]


def _node_mlp(x, h, mp0, mp1, fp0, fp1, w1a, w1b, b1, w2, b2):
    n, hdim = h.shape
    bn = 1000
    assert n % bn == 0
    full = lambda a: pl.BlockSpec(a.shape, lambda i: (0,) * a.ndim)
    return pl.pallas_call(
        _node_body,
        grid=(n // bn,),
        in_specs=[
            pl.BlockSpec((bn, x.shape[1]), lambda i: (i, 0)),
            pl.BlockSpec((bn, hdim), lambda i: (i, 0)),
            pl.BlockSpec((bn, 128), lambda i: (i, 0)),
            pl.BlockSpec((bn, 128), lambda i: (i, 0)),
            pl.BlockSpec((bn, 128), lambda i: (i, 0)),
            pl.BlockSpec((bn, 128), lambda i: (i, 0)),
            full(w1a), full(w1b), full(b1), full(w2), full(b2),
        ],
        out_specs=[
            pl.BlockSpec((bn, x.shape[1]), lambda i: (i, 0)),
            pl.BlockSpec((bn, hdim), lambda i: (i, 0)),
        ],
        out_shape=[
            jax.ShapeDtypeStruct((n, x.shape[1]), jnp.float32),
            jax.ShapeDtypeStruct((n, hdim), jnp.float32),
        ],
    )(x, h, mp0, mp1, fp0, fp1, w1a, w1b, b1, w2, b2)


def kernel(x, h, edge_index, edge_fea,
           em_W1, em_b1, em_W2, em_b2,
           co_W1, co_b1, co_W2, co_b2,
           nn_W1, nn_b1, nn_W2, nn_b2):
    n, hdim = h.shape
    row = edge_index[0].astype(jnp.int32)
    col = edge_index[1].astype(jnp.int32)

    w1s = em_W1[0:1, :]
    w1r = em_W1[1:1 + hdim, :]
    w1c = em_W1[1 + hdim:1 + 2 * hdim, :]
    w1e = em_W1[1 + 2 * hdim:, :]

    a_tab, b_tab = _build_tables(x, h, w1r, w1c, em_b1.reshape(1, -1))

    e = row.shape[0]
    eh = e // 2
    mps, fps = [], []
    edge_args = (w1e, w1s, em_W2, em_b2.reshape(1, -1),
                 co_W1, co_b1.reshape(1, -1), co_W2, co_b2.reshape(1, 1))
    for half in range(2):
        sl = slice(half * eh, (half + 1) * eh)
        g1_h, g2_h = _sc_gather(a_tab, b_tab, row[sl], col[sl], 40)
        msg_h, faux_h = _edge_mlp(g1_h, g2_h, edge_fea[sl], *edge_args)
        mp_h, fp_h = _sc_scatter(row[sl], msg_h, faux_h, n)
        mps.append(mp_h)
        fps.append(fp_h)
    x_new, h_new = _node_mlp(
        x, h, mps[0][:n], mps[1][:n], fps[0][:n], fps[1][:n],
        nn_W1[:hdim], nn_W1[hdim:], nn_b1.reshape(1, -1),
        nn_W2, nn_b2.reshape(1, -1))
    return (x_new, h_new)
